# baseline (speedup 1.0000x reference)
import functools

import jax
import jax.numpy as jnp
from jax import lax
from jax.experimental import pallas as pl
from jax.experimental.pallas import tpu as pltpu
from jax.experimental.pallas import tpu_sc as plsc

_MIN_VAL = 0.0
_MAX_VAL = 1.0
_COUNT = 100000
_DIM = 64
_BATCH = 16384

_NC, _NS, _L = 2, 16, 16
_NW = _NC * _NS
_FPW = _DIM // _NW
_CHUNK = 4096
_NXC = _BATCH // _CHUNK
_NOC = _BATCH // _CHUNK

_mesh = plsc.VectorSubcoreMesh(core_axis_name="c", subcore_axis_name="s")


@functools.partial(
    pl.kernel,
    mesh=_mesh,
    out_type=jax.ShapeDtypeStruct((_DIM, _BATCH), jnp.float32),
    scratch_types=[
        pltpu.VMEM((_COUNT,), jnp.float32),
        pltpu.VMEM((_BATCH,), jnp.int32),
        pltpu.VMEM((2, _CHUNK), jnp.float32),
        pltpu.SemaphoreType.DMA,
        pltpu.SemaphoreType.DMA,
        pltpu.SemaphoreType.DMA,
        pltpu.SemaphoreType.DMA,
    ],
    compiler_params=pltpu.CompilerParams(
        use_tc_tiling_on_sc=True, needs_layout_passes=False
    ),
)
def _bucketed_gather(
    x_hbm, tab_hbm, out_hbm,
    row_v, idx_v, buf_v, sem_r0, sem_r1, sem_b0, sem_b1,
):
    wid = lax.axis_index("s") * _NC + lax.axis_index("c")
    f_base = wid * _FPW

    def start_row(f):
        return [pltpu.async_copy(tab_hbm.at[f], row_v, sem_r0)]

    bsems = (sem_b0, sem_b1)
    xcopies = {}
    for c in range(min(2, _NXC)):
        xcopies[c] = pltpu.async_copy(
            x_hbm.at[pl.ds(c * _CHUNK, _CHUNK)], buf_v.at[c], bsems[c]
        )
    row_copy = start_row(f_base)

    scale = float(_COUNT) / (_MAX_VAL - _MIN_VAL)
    for c in range(_NXC):
        xcopies.pop(c).wait()

        @plsc.parallel_loop(0, _CHUNK // _L, unroll=8)
        def compute_idx(g, c=c):
            v = buf_v[c % 2, pl.ds(g * _L, _L)]
            scaled = (v - _MIN_VAL) * scale
            clipped = jnp.clip(scaled, 0.0, float(_COUNT - 1))
            idx_v[pl.ds(c * _CHUNK + g * _L, _L)] = clipped.astype(jnp.int32)

        if c + 2 < _NXC:
            xcopies[c + 2] = pltpu.async_copy(
                x_hbm.at[pl.ds((c + 2) * _CHUNK, _CHUNK)],
                buf_v.at[c % 2],
                bsems[c % 2],
            )

    owrites = {}
    for fi in range(_FPW):
        for rc in row_copy:
            rc.wait()

        for oc in range(_NOC):
            b = oc % 2
            prev = owrites.pop(b, None)
            if prev is not None:
                prev.wait()

            @plsc.parallel_loop(0, _CHUNK // _L, unroll=16)
            def gather_chunk(g, oc=oc, b=b):
                iv = idx_v[pl.ds(oc * _CHUNK + g * _L, _L)]
                buf_v[b, pl.ds(g * _L, _L)] = plsc.load_gather(row_v, [iv])

            owrites[b] = pltpu.async_copy(
                buf_v.at[b],
                out_hbm.at[f_base + fi, pl.ds(oc * _CHUNK, _CHUNK)],
                bsems[b],
            )

        if fi + 1 < _FPW:
            row_copy = start_row(f_base + fi + 1)

    for prev in owrites.values():
        prev.wait()


def kernel(x, table):
    out_t = _bucketed_gather(x, table.T)
    return out_t.T
